# baseline (device time: 401078 ns/iter reference)
import jax
import jax.numpy as jnp
from jax import lax
from jax.experimental import pallas as pl
from jax.experimental.pallas import tpu as pltpu

N_DEV = 32
M, K_SHARD, N = 4096, 128, 2048
CHUNK = M // N_DEV
N_RINGS = 8
QW = N // N_RINGS
N_STEPS = 2 * (N_DEV - 1)

_P = [(0, 0), (1, 0), (2, 0), (3, 0),
      (3, 1), (2, 1), (1, 1), (0, 1),
      (0, 2), (1, 2), (2, 2), (3, 2),
      (3, 3), (2, 3), (1, 3), (0, 3)]
_RING_COORDS = [(0, y, z) for (y, z) in _P] + [(1, y, z) for (y, z) in reversed(_P)]

_PLANE_IDX = {(0, 0): 0, (1, 0): 1, (1, 1): 2, (0, 1): 3,
              (0, 2): 4, (1, 2): 5, (1, 3): 6, (0, 3): 7}
_RING = [8 * z + _PLANE_IDX[(x, y)] for (x, y, z) in _RING_COORDS]
_POS = [0] * N_DEV
for _r, _l in enumerate(_RING):
    _POS[_l] = _r
assert sorted(_RING) == list(range(N_DEV))


def kernel(x, w_mat, scale_x, scale_w):
    my = lax.axis_index("i")
    pos = jnp.asarray(_POS, jnp.int32)[my]
    ring = jnp.asarray(_RING, jnp.int32)
    right = ring[lax.rem(pos + 1, N_DEV)]
    left = ring[lax.rem(pos + N_DEV - 1, N_DEV)]
    pos_a = pos.reshape(1)
    left_a = left.reshape(1)
    right_a = right.reshape(1)

    def body(x_ref, w_ref, sx_ref, sw_ref, pos_ref, left_ref, right_ref,
             out_ref, bufs_ref, send_sems, recv_sems, credits):
        pos = pos_ref[0]
        left = left_ref[0]
        right = right_ref[0]

        barrier_sem = pltpu.get_barrier_semaphore()
        for nbr in (left, right):
            pl.semaphore_signal(barrier_sem, inc=1, device_id=(nbr,),
                                device_id_type=pl.DeviceIdType.MESH)
        pl.semaphore_wait(barrier_sem, 2)

        acc = lax.dot_general(
            x_ref[...], w_ref[...], (((1,), (0,)), ((), ())),
            preferred_element_type=jnp.int32)
        out_ref[...] = acc.astype(jnp.float32)

        cw = tuple(k < N_RINGS // 2 for k in range(N_RINGS))
        s = sx_ref[0] * sw_ref[0]

        def qslice(k, c):
            return (pl.ds(c * CHUNK, CHUNK), pl.ds(k * QW, QW))

        def chunk_of(k, h):
            if h < N_DEV - 1:
                d = (2 * N_DEV - h) if cw[k] else h
            else:
                g = h - (N_DEV - 1)
                d = (2 * N_DEV + 1 - g) if cw[k] else (N_DEV - 1 + g)
            return lax.rem(pos + d, N_DEV)

        def make_rdma(k, h):
            c = chunk_of(k, h)
            if h < N_DEV - 1:
                dst = bufs_ref.at[k, (h + 1) % 2]
            else:
                dst = out_ref.at[qslice(k, c)]
            return pltpu.make_async_remote_copy(
                src_ref=out_ref.at[qslice(k, c)],
                dst_ref=dst,
                send_sem=send_sems.at[k, h % 2],
                recv_sem=recv_sems.at[k, (h + 1) % 2],
                device_id=(right if cw[k] else left,),
                device_id_type=pl.DeviceIdType.MESH)

        def consume(k, h):
            if h < N_DEV - 1:
                c = lax.rem(pos + ((2 * N_DEV - h - 1) if cw[k] else (h + 1)),
                            N_DEV)
                sl = qslice(k, c)
                out_ref[sl] = out_ref[sl] + bufs_ref[k, (h + 1) % 2]
                if h == N_DEV - 2:
                    own = lax.rem(pos + (1 if cw[k] else N_DEV - 1), N_DEV)
                    osl = qslice(k, own)
                    y = out_ref[osl] * s
                    out_ref[osl] = y / (1.0 + jnp.exp(-y))

        def send_credit(k, h):
            if h < N_STEPS - 2:
                pl.semaphore_signal(credits.at[k], inc=1,
                                    device_id=(left if cw[k] else right,),
                                    device_id_type=pl.DeviceIdType.MESH)

        inflight = [make_rdma(k, 0) for k in range(N_RINGS)]
        for k in range(N_RINGS):
            inflight[k].start()
        for h in range(N_STEPS):
            for k in range(N_RINGS):
                inflight[k].wait()
                consume(k, h)
                send_credit(k, h)
                if h + 1 < N_STEPS:
                    if h + 1 >= 2:
                        pl.semaphore_wait(credits.at[k], 1)
                    nxt = make_rdma(k, h + 1)
                    nxt.start()
                    inflight[k] = nxt

    return pl.pallas_call(
        body,
        out_shape=jax.ShapeDtypeStruct((M, N), jnp.float32),
        in_specs=[
            pl.BlockSpec(memory_space=pltpu.VMEM),
            pl.BlockSpec(memory_space=pltpu.VMEM),
            pl.BlockSpec(memory_space=pltpu.SMEM),
            pl.BlockSpec(memory_space=pltpu.SMEM),
            pl.BlockSpec(memory_space=pltpu.SMEM),
            pl.BlockSpec(memory_space=pltpu.SMEM),
            pl.BlockSpec(memory_space=pltpu.SMEM),
        ],
        out_specs=pl.BlockSpec(memory_space=pltpu.VMEM),
        scratch_shapes=[
            pltpu.VMEM((N_RINGS, 2, CHUNK, QW), jnp.float32),
            pltpu.SemaphoreType.DMA((N_RINGS, 2)),
            pltpu.SemaphoreType.DMA((N_RINGS, 2)),
            pltpu.SemaphoreType.REGULAR((N_RINGS,)),
        ],
        compiler_params=pltpu.CompilerParams(
            collective_id=0,
            vmem_limit_bytes=100 * 1024 * 1024,
        ),
    )(x, w_mat, scale_x, scale_w, pos_a, left_a, right_a)
